# baseline (speedup 1.0000x reference)
import functools

import jax
import jax.numpy as jnp
from jax import lax
from jax.experimental import pallas as pl
from jax.experimental.pallas import tpu as pltpu
from jax.experimental.pallas import tpu_sc as plsc

NC, NS, LANES = 2, 16, 16
NW = NC * NS
B = 4096
DIM = 64
NEG = 20
VOCAB = 1000000
BPW = B // NW
WU_IDX = BPW * (1 + NEG)
NSLOT = 2
GROUPS = DIM // NSLOT

_MESH = plsc.VectorSubcoreMesh(core_axis_name="c", subcore_axis_name="s")
_PARAMS = pltpu.CompilerParams(use_tc_tiling_on_sc=False)


def _sc_gather(WvT, WuT, civ_hbm, riv_hbm):
    out_t = [jax.ShapeDtypeStruct((NW * DIM, BPW), jnp.float32)] * 3

    @functools.partial(
        pl.kernel,
        out_type=out_t,
        mesh=_MESH,
        compiler_params=_PARAMS,
        scratch_types=[
            pltpu.VMEM((BPW,), jnp.int32),
            pltpu.VMEM((WU_IDX,), jnp.int32),
            pltpu.VMEM((2 * NSLOT, BPW), jnp.float32),
            pltpu.VMEM((2 * NSLOT, WU_IDX), jnp.float32),
            pltpu.VMEM((DIM, BPW), jnp.float32),
            pltpu.VMEM((DIM, BPW), jnp.float32),
            pltpu.VMEM((DIM, BPW), jnp.float32),
        ]
        + [pltpu.SemaphoreType.DMA] * (4 * NSLOT),
    )
    def k(wv_hbm, wu_hbm, c_hbm, r_hbm, oc_hbm, ot_hbm, on_hbm,
          civ, riv, cg, wg, oc, ot, on, *sems):
        csem = sems[:2 * NSLOT]
        wsem = sems[2 * NSLOT:]
        wid = lax.axis_index("c") * NS + lax.axis_index("s")

        pltpu.sync_copy(c_hbm.at[wid], civ)
        pltpu.sync_copy(r_hbm.at[wid], riv)

        def issue(g, half):
            hs = []
            for cc in range(NSLOT):
                slot = half * NSLOT + cc
                c = g * NSLOT + cc
                hs.append((
                    pltpu.async_copy(wv_hbm.at[c].at[civ], cg.at[slot],
                                     csem[slot]),
                    pltpu.async_copy(wu_hbm.at[c].at[riv], wg.at[slot],
                                     wsem[slot]),
                ))
            return hs

        pend = {0: issue(0, 0)}
        for g in range(GROUPS):
            if g + 1 < GROUPS:
                pend[g + 1] = issue(g + 1, (g + 1) % 2)
            for cc in range(NSLOT):
                slot = (g % 2) * NSLOT + cc
                c = g * NSLOT + cc
                hc, hw = pend[g][cc]
                hc.wait()
                hw.wait()

                @pl.loop(0, BPW // LANES)
                def _(kk, slot=slot, c=c):
                    oc[c, pl.ds(kk * LANES, LANES)] = (
                        cg[slot, pl.ds(kk * LANES, LANES)])
                    ot[c, pl.ds(kk * LANES, LANES)] = (
                        wg[slot, pl.ds(kk * LANES, LANES)])
                    on[c, pl.ds(kk * LANES, LANES)] = (
                        wg[slot, pl.ds(BPW + kk * LANES, LANES)])

                @pl.loop(1, NEG)
                def _(j, slot=slot, c=c):
                    @pl.loop(0, BPW // LANES)
                    def _(kk):
                        plsc.addupdate(
                            on.at[c, pl.ds(kk * LANES, LANES)],
                            wg[slot, pl.ds((1 + j) * BPW + kk * LANES,
                                           LANES)])
            del pend[g]

        base = wid * DIM
        pltpu.sync_copy(oc, oc_hbm.at[pl.ds(base, DIM)])
        pltpu.sync_copy(ot, ot_hbm.at[pl.ds(base, DIM)])
        pltpu.sync_copy(on, on_hbm.at[pl.ds(base, DIM)])

    return k(WvT, WuT, civ_hbm, riv_hbm)


def _tc_loss(ce, te, ns):
    def body(c_ref, t_ref, n_ref, o_ref):
        c = c_ref[...]
        t = t_ref[...]
        n = n_ref[...]
        pos = jnp.sum((c * t).reshape(NW, DIM, BPW), axis=1)
        neg = -jnp.sum((c * n).reshape(NW, DIM, BPW), axis=1)

        def logsig(x):
            return jnp.minimum(x, 0.0) - jnp.log1p(jnp.exp(-jnp.abs(x)))

        tot = jnp.sum(logsig(pos)) + jnp.sum(logsig(neg))
        o_ref[...] = jnp.reshape(-tot / B, (1, 1))

    return pl.pallas_call(
        body,
        out_shape=jax.ShapeDtypeStruct((1, 1), jnp.float32),
    )(ce, te, ns)


def kernel(center_words, target_words, negative_words, Wv, Wu):
    WvT = jnp.transpose(Wv)
    WuT = jnp.transpose(Wu)
    civ = center_words.astype(jnp.int32).reshape(NW, BPW)
    tgt = target_words.astype(jnp.int32).reshape(NW, BPW)
    neg = jnp.transpose(
        negative_words.astype(jnp.int32).reshape(NW, BPW, NEG), (0, 2, 1))
    riv = jnp.concatenate([tgt[:, None, :], neg], axis=1).reshape(NW, WU_IDX)
    ce, te, nsum = _sc_gather(WvT, WuT, civ, riv)
    out = _tc_loss(ce, te, nsum)
    return jnp.reshape(out, ())
